# baseline (speedup 1.0000x reference)
import functools

import jax
import jax.numpy as jnp
from jax.experimental import pallas as pl
from jax.experimental.pallas import tpu as pltpu

_K = 64
_UNGATED = 12
_GROUP = 4


def _matmul_into(x_ref, z_ref, w_ref, s_ref, buf):
    upd = jax.lax.dot_general(
        x_ref[...], w_ref[...], (((1,), (1,)), ((), ())),
        preferred_element_type=jnp.float32)
    upd = upd + jax.lax.dot_general(
        z_ref[...], s_ref[...], (((1,), (1,)), ((), ())),
        preferred_element_type=jnp.float32)
    buf[...] = upd


def _radix_pass(bits, t, cnt_t, b):
    cand = t | jnp.int32(1 << b)
    cnt = jnp.sum((bits >= cand).astype(jnp.int32), axis=1, keepdims=True)
    ge = cnt >= _K
    return jnp.where(ge, cand, t), jnp.where(ge, cnt, cnt_t)


def _select_store(buf, o_ref, t_ref, c_ref):
    upd = buf[...]
    bits = jax.lax.bitcast_convert_type(upd, jnp.int32) & jnp.int32(0x7FFFFFFF)
    rows = upd.shape[0]
    t = jnp.zeros((rows, 1), jnp.int32)
    cnt_t = jnp.full((rows, 1), upd.shape[1], jnp.int32)
    bit = 30
    for _ in range(_UNGATED):
        t, cnt_t = _radix_pass(bits, t, cnt_t, bit)
        bit -= 1
    t_ref[...] = t
    c_ref[...] = cnt_t
    while bit >= 0:
        nbits = min(_GROUP, bit + 1)
        first = bit

        @pl.when(jnp.any(c_ref[...] != _K))
        def _(first=first, nbits=nbits):
            t = t_ref[...]
            cnt_t = c_ref[...]
            for b in range(first, first - nbits, -1):
                t, cnt_t = _radix_pass(bits, t, cnt_t, b)
            t_ref[...] = t
            c_ref[...] = cnt_t

        bit -= nbits
    o_ref[...] = jnp.where(bits >= t_ref[...], upd, 0.0)


def _pipelined_block(x_ref, z_ref, w_ref, s_ref, o_ref, buf, t_ref, c_ref, *,
                     nblocks):
    i = pl.program_id(0)
    _select_store(buf.at[(i + 1) % 2], o_ref, t_ref, c_ref)
    _matmul_into(x_ref, z_ref, w_ref, s_ref, buf.at[i % 2])


@functools.partial(jax.jit, static_argnames=("block_rows",))
def kernel(x, z_prev, W, S, block_rows: int = 256):
    batch, input_dim = x.shape
    code_dim = W.shape[0]
    nblocks = batch // block_rows
    grid = (nblocks + 1,)
    return pl.pallas_call(
        functools.partial(_pipelined_block, nblocks=nblocks),
        grid=grid,
        in_specs=[
            pl.BlockSpec((block_rows, input_dim),
                         lambda i: (jnp.minimum(i, nblocks - 1), 0)),
            pl.BlockSpec((block_rows, code_dim),
                         lambda i: (jnp.minimum(i, nblocks - 1), 0)),
            pl.BlockSpec((code_dim, input_dim), lambda i: (0, 0)),
            pl.BlockSpec((code_dim, code_dim), lambda i: (0, 0)),
        ],
        out_specs=pl.BlockSpec((block_rows, code_dim),
                               lambda i: (jnp.maximum(i - 1, 0), 0)),
        out_shape=jax.ShapeDtypeStruct((batch, code_dim), jnp.float32),
        scratch_shapes=[
            pltpu.VMEM((2, block_rows, code_dim), jnp.float32),
            pltpu.VMEM((block_rows, 1), jnp.int32),
            pltpu.VMEM((block_rows, 1), jnp.int32),
        ],
    )(x, z_prev, W, S)


# speedup vs baseline: 1.0894x; 1.0894x over previous
import functools

import jax
import jax.numpy as jnp
from jax.experimental import pallas as pl
from jax.experimental.pallas import tpu as pltpu

_K = 64
_UNGATED = 12
_GROUP = 4


def _matmul_into(x_ref, z_ref, w_ref, s_ref, buf):
    upd = jax.lax.dot_general(
        x_ref[...], w_ref[...], (((1,), (1,)), ((), ())),
        preferred_element_type=jnp.float32)
    upd = upd + jax.lax.dot_general(
        z_ref[...], s_ref[...], (((1,), (1,)), ((), ())),
        preferred_element_type=jnp.float32)
    buf[...] = upd


def _radix_pass(bits, t, cnt_t, b):
    cand = t | jnp.int32(1 << b)
    cnt = jnp.sum((bits >= cand).astype(jnp.int32), axis=1, keepdims=True)
    ge = cnt >= _K
    return jnp.where(ge, cand, t), jnp.where(ge, cnt, cnt_t)


def _select_store(buf, o_ref, t_ref, c_ref):
    upd = buf[...]
    bits = jax.lax.bitcast_convert_type(upd, jnp.int32) & jnp.int32(0x7FFFFFFF)
    rows = upd.shape[0]
    t = jnp.zeros((rows, 1), jnp.int32)
    cnt_t = jnp.full((rows, 1), upd.shape[1], jnp.int32)
    for b in range(30, -1, -1):
        t, cnt_t = _radix_pass(bits, t, cnt_t, b)
    o_ref[...] = jnp.where(bits >= t, upd, 0.0)


def _pipelined_block(x_ref, z_ref, w_ref, s_ref, o_ref, buf, t_ref, c_ref, *,
                     nblocks):
    i = pl.program_id(0)
    _select_store(buf.at[(i + 1) % 2], o_ref, t_ref, c_ref)
    _matmul_into(x_ref, z_ref, w_ref, s_ref, buf.at[i % 2])


@functools.partial(jax.jit, static_argnames=("block_rows",))
def kernel(x, z_prev, W, S, block_rows: int = 256):
    batch, input_dim = x.shape
    code_dim = W.shape[0]
    nblocks = batch // block_rows
    grid = (nblocks + 1,)
    return pl.pallas_call(
        functools.partial(_pipelined_block, nblocks=nblocks),
        grid=grid,
        in_specs=[
            pl.BlockSpec((block_rows, input_dim),
                         lambda i: (jnp.minimum(i, nblocks - 1), 0)),
            pl.BlockSpec((block_rows, code_dim),
                         lambda i: (jnp.minimum(i, nblocks - 1), 0)),
            pl.BlockSpec((code_dim, input_dim), lambda i: (0, 0)),
            pl.BlockSpec((code_dim, code_dim), lambda i: (0, 0)),
        ],
        out_specs=pl.BlockSpec((block_rows, code_dim),
                               lambda i: (jnp.maximum(i - 1, 0), 0)),
        out_shape=jax.ShapeDtypeStruct((batch, code_dim), jnp.float32),
        scratch_shapes=[
            pltpu.VMEM((2, block_rows, code_dim), jnp.float32),
            pltpu.VMEM((block_rows, 1), jnp.int32),
            pltpu.VMEM((block_rows, 1), jnp.int32),
        ],
    )(x, z_prev, W, S)
